# Initial kernel scaffold; baseline (speedup 1.0000x reference)
import functools

import jax
import jax.numpy as jnp
from jax import lax
from jax.experimental import pallas as pl
from jax.experimental.pallas import tpu as pltpu
from jax.experimental.pallas import tpu_sc as plsc

_f32 = jnp.float32
_i32 = jnp.int32

N = 10000
NP = 10240
E = 60000
EP = 61440
NW = 32
VPW = NP // NW
EPW = EP // NW
ECH = 128
CV = 32
ROWS_PER_TILE = NP // 16

LVL = [(56, 0), (28, 3136), (14, 3920), (7, 4120)]
T_ROWS = 4176

_MESH = plsc.VectorSubcoreMesh(core_axis_name="c", subcore_axis_name="s",
                               num_cores=2, num_subcores=16)


def _gbuild(f0t, f1t, f2t, f3t, w_lin, b_lin):
    def body(f0_ref, f1_ref, f2_ref, f3_ref, w_ref, b_ref, out_ref):
        qb = b_ref[...] * 0.25
        out_ref[0:3136, :] = jnp.dot(f0_ref[...], w_ref[0:256, :],
                                     preferred_element_type=_f32) + qb
        out_ref[3136:3920, :] = jnp.dot(f1_ref[...], w_ref[256:768, :],
                                        preferred_element_type=_f32) + qb
        out_ref[3920:4116, :] = jnp.dot(f2_ref[...], w_ref[768:1792, :],
                                        preferred_element_type=_f32) + qb
        out_ref[4120:4169, :] = jnp.dot(f3_ref[...], w_ref[1792:3840, :],
                                        preferred_element_type=_f32) + qb

    return pl.pallas_call(
        body,
        out_shape=jax.ShapeDtypeStruct((T_ROWS, 128), _f32),
    )(f0t, f1t, f2t, f3t, w_lin, b_lin)


def _align_body(g_hbm, px_hbm, py_hbm, pz_hbm, out_hbm,
                xv, yv, zv, idxb, wb, rows, outb, sem):
    c = lax.axis_index("c")
    s = lax.axis_index("s")
    wid = s * 2 + c
    vbase = wid * VPW
    pltpu.sync_copy(px_hbm.at[pl.ds(vbase, VPW)], xv)
    pltpu.sync_copy(py_hbm.at[pl.ds(vbase, VPW)], yv)
    pltpu.sync_copy(pz_hbm.at[pl.ds(vbase, VPW)], zv)

    def build(k, carry):
        sl = pl.ds(k * 16, 16)
        x = xv[sl]
        y = yv[sl]
        z = zv[sl]
        d = jnp.abs(z) + 2.0
        u = jnp.clip(x / d, -1.0, 1.0)
        v = jnp.clip(y / d, -1.0, 1.0)
        for l, (wl, off) in enumerate(LVL):
            fw = float(wl - 1)
            pxx = (u + 1.0) * 0.5 * fw
            pyy = (v + 1.0) * 0.5 * fw
            x0 = pxx.astype(_i32)
            y0 = pyy.astype(_i32)
            dx = jnp.minimum(x0 + 1, wl - 1) - x0
            y1 = jnp.minimum(y0 + 1, wl - 1)
            wx = pxx - x0.astype(_f32)
            wy = pyy - y0.astype(_f32)
            r0 = y0 * wl + (x0 + off)
            r1 = y1 * wl + (x0 + off)
            idxb[4 * l + 0, sl] = r0
            idxb[4 * l + 1, sl] = r0 + dx
            idxb[4 * l + 2, sl] = r1
            idxb[4 * l + 3, sl] = r1 + dx
            wb[4 * l + 0, sl] = (1.0 - wx) * (1.0 - wy)
            wb[4 * l + 1, sl] = wx * (1.0 - wy)
            wb[4 * l + 2, sl] = (1.0 - wx) * wy
            wb[4 * l + 3, sl] = wx * wy
        return carry

    lax.fori_loop(0, VPW // 16, build, 0)

    for ci in range(VPW // CV):
        cb = ci * CV
        handles = [
            pltpu.async_copy(g_hbm.at[idxb.at[t, pl.ds(cb, CV)]], rows.at[t], sem)
            for t in range(16)
        ]
        for h in handles:
            h.wait()

        def accum(i, carry):
            accs = [jnp.zeros((16,), _f32) for _ in range(8)]
            col = jnp.full((16,), cb + i, _i32)
            for t in range(16):
                wsp = plsc.load_gather(wb, [jnp.full((16,), t, _i32), col])
                for b in range(8):
                    accs[b] = accs[b] + wsp * rows[t, i, pl.ds(b * 16, 16)]
            for b in range(8):
                outb[i, pl.ds(b * 16, 16)] = accs[b]
            return carry

        lax.fori_loop(0, CV, accum, 0)
        pltpu.sync_copy(outb, out_hbm.at[pl.ds(vbase + cb, CV)])


_align = functools.partial(
    pl.kernel,
    _align_body,
    out_type=jax.ShapeDtypeStruct((NP, 128), _f32),
    mesh=_MESH,
    scratch_types=[
        pltpu.VMEM((VPW,), _f32),
        pltpu.VMEM((VPW,), _f32),
        pltpu.VMEM((VPW,), _f32),
        pltpu.VMEM((16, VPW), _i32),
        pltpu.VMEM((16, VPW), _f32),
        pltpu.VMEM((16, CV, 128), _f32),
        pltpu.VMEM((CV, 128), _f32),
        pltpu.SemaphoreType.DMA,
    ],
)()


def _deg_body(dstp_hbm, ones_hbm, zeros_hbm, out_hbm, didx, ones_v, acc_sh, sem):
    c = lax.axis_index("c")
    s = lax.axis_index("s")
    wid = s * 2 + c
    pltpu.sync_copy(ones_hbm, ones_v)
    pltpu.sync_copy(zeros_hbm,
                    acc_sh.at[pl.ds(s * ROWS_PER_TILE, ROWS_PER_TILE)])
    plsc.subcore_barrier()
    base = wid * EPW
    for k in range(EPW // ECH):
        pltpu.sync_copy(dstp_hbm.at[pl.ds(base + k * ECH, ECH)], didx)
        pltpu.sync_copy(ones_v, acc_sh.at[didx], add=True)
    plsc.subcore_barrier()
    pltpu.sync_copy(acc_sh.at[pl.ds(s * ROWS_PER_TILE, ROWS_PER_TILE)],
                    out_hbm.at[c, pl.ds(s * ROWS_PER_TILE, ROWS_PER_TILE)])


_deg = functools.partial(
    pl.kernel,
    _deg_body,
    out_type=jax.ShapeDtypeStruct((2, NP, 16), _f32),
    mesh=_MESH,
    scratch_types=[
        pltpu.VMEM((ECH,), _i32),
        pltpu.VMEM((ECH, 16), _f32),
        pltpu.VMEM_SHARED((NP, 16), _f32),
        pltpu.SemaphoreType.DMA,
    ],
)()


def _segsum_body(z_hbm, srcp_hbm, dstp_hbm, zeros_hbm, out_hbm,
                 sidx, didx, rows, acc_sh, sem):
    c = lax.axis_index("c")
    s = lax.axis_index("s")
    wid = s * 2 + c
    pltpu.sync_copy(zeros_hbm,
                    acc_sh.at[pl.ds(s * ROWS_PER_TILE, ROWS_PER_TILE)])
    plsc.subcore_barrier()
    base = wid * EPW
    for k in range(EPW // ECH):
        pltpu.sync_copy(srcp_hbm.at[pl.ds(base + k * ECH, ECH)], sidx)
        pltpu.async_copy(z_hbm.at[sidx], rows, sem).wait()
        pltpu.sync_copy(dstp_hbm.at[pl.ds(base + k * ECH, ECH)], didx)
        pltpu.sync_copy(rows, acc_sh.at[didx], add=True)
    plsc.subcore_barrier()
    pltpu.sync_copy(acc_sh.at[pl.ds(s * ROWS_PER_TILE, ROWS_PER_TILE)],
                    out_hbm.at[c, pl.ds(s * ROWS_PER_TILE, ROWS_PER_TILE)])


_segsum = functools.partial(
    pl.kernel,
    _segsum_body,
    out_type=jax.ShapeDtypeStruct((2, NP, 128), _f32),
    mesh=_MESH,
    scratch_types=[
        pltpu.VMEM((ECH,), _i32),
        pltpu.VMEM((ECH,), _i32),
        pltpu.VMEM((ECH, 128), _f32),
        pltpu.VMEM_SHARED((NP, 128), _f32),
        pltpu.SemaphoreType.DMA,
    ],
)()


_BM = 2048


def _t1(vf, w, b):
    def body(vf_ref, w_ref, b_ref, a_ref, z_ref, s_ref):
        az = jnp.dot(vf_ref[...], w_ref[...],
                     preferred_element_type=_f32) + b_ref[...]
        a_ref[...] = az[:, 0:128]
        z_ref[...] = az[:, 128:256]
        s_ref[...] = az[:, 256:384]

    return pl.pallas_call(
        body,
        grid=(NP // _BM,),
        in_specs=[
            pl.BlockSpec((_BM, 384), lambda i: (i, 0)),
            pl.BlockSpec((384, 384), lambda i: (0, 0)),
            pl.BlockSpec((1, 384), lambda i: (0, 0)),
        ],
        out_specs=[pl.BlockSpec((_BM, 128), lambda i: (i, 0))] * 3,
        out_shape=[jax.ShapeDtypeStruct((NP, 128), _f32)] * 3,
    )(vf, w, b)


def _nbr_of(p0, p1, d0, d1):
    deg = jnp.maximum(d0[:, 0:1] + d1[:, 0:1], 1.0)
    return (p0 + p1) / deg


def _tmid(aprev, p0, p1, d0, d1, resid, w, b, *, relu, emit_x):
    has_resid = resid is not None

    def body(*refs):
        if has_resid:
            (ap_ref, p0_ref, p1_ref, d0_ref, d1_ref, r_ref, w_ref, b_ref,
             *outs) = refs
        else:
            (ap_ref, p0_ref, p1_ref, d0_ref, d1_ref, w_ref, b_ref,
             *outs) = refs
        x = ap_ref[...] + _nbr_of(p0_ref[...], p1_ref[...],
                                  d0_ref[...], d1_ref[...])
        if relu:
            x = jnp.maximum(x, 0.0)
        if has_resid:
            x = x + r_ref[...]
        az = jnp.dot(x, w_ref[...], preferred_element_type=_f32) + b_ref[...]
        if emit_x:
            outs[0][...] = x
            outs = outs[1:]
        outs[0][...] = az[:, 0:128]
        outs[1][...] = az[:, 128:256]

    row = pl.BlockSpec((_BM, 128), lambda i: (i, 0))
    deg_spec = pl.BlockSpec((_BM, 16), lambda i: (i, 0))
    in_specs = [row, row, row, deg_spec, deg_spec]
    args = [aprev, p0, p1, d0, d1]
    if has_resid:
        in_specs.append(row)
        args.append(resid)
    in_specs += [pl.BlockSpec((128, 256), lambda i: (0, 0)),
                 pl.BlockSpec((1, 256), lambda i: (0, 0))]
    args += [w, b]
    n_out = 3 if emit_x else 2
    return pl.pallas_call(
        body,
        grid=(NP // _BM,),
        in_specs=in_specs,
        out_specs=[row] * n_out,
        out_shape=[jax.ShapeDtypeStruct((NP, 128), _f32)] * n_out,
    )(*args)


def _tfinal(pos_pad, a, p0, p1, d0, d1):
    def body(pos_ref, a_ref, p0_ref, p1_ref, d0_ref, d1_ref, out_ref):
        nbr = _nbr_of(p0_ref[...], p1_ref[...], d0_ref[...], d1_ref[...])
        out_ref[...] = pos_ref[...] + jnp.tanh(a_ref[...] + nbr)

    row = pl.BlockSpec((_BM, 128), lambda i: (i, 0))
    deg_spec = pl.BlockSpec((_BM, 16), lambda i: (i, 0))
    return pl.pallas_call(
        body,
        grid=(NP // _BM,),
        in_specs=[row, row, row, row, deg_spec, deg_spec],
        out_specs=row,
        out_shape=jax.ShapeDtypeStruct((NP, 128), _f32),
    )(pos_pad, a, p0, p1, d0, d1)


def kernel(conv2_3, conv3_4, conv4_6, conv5_3, vertex_positions,
           vertex_features, edge_index, params):
    p = params

    f0t = conv2_3.reshape(256, 3136).T
    f1t = conv3_4.reshape(512, 784).T
    f2t = conv4_6.reshape(1024, 196).T
    f3t = conv5_3.reshape(2048, 49).T
    g = _gbuild(f0t, f1t, f2t, f3t, p['W_lin'], p['b_lin'].reshape(1, 128))

    pos_p = jnp.pad(vertex_positions, ((0, NP - N), (0, 0)))
    proj = _align(g, pos_p[:, 0], pos_p[:, 1], pos_p[:, 2])

    srcp = jnp.concatenate([edge_index[0], jnp.zeros((EP - E,), _i32)])
    dstp = jnp.concatenate([edge_index[1], jnp.full((EP - E,), NP - 1, _i32)])
    ones_e = jnp.ones((ECH, 16), _f32)
    zeros16 = jnp.zeros((ROWS_PER_TILE, 16), _f32)
    zeros128 = jnp.zeros((ROWS_PER_TILE, 128), _f32)

    degp = _deg(dstp, ones_e, zeros16)
    d0, d1 = degp[0], degp[1]

    vf_p = jnp.pad(vertex_features, ((0, NP - N), (0, 0)))
    vf = jnp.concatenate(
        [vf_p, pos_p, proj, jnp.zeros((NP, 125), _f32)], axis=1)

    padw = lambda w: jnp.pad(w, ((0, 384 - w.shape[0]), (0, 0)))
    w1 = jnp.concatenate(
        [padw(p['rg0_w0a']), padw(p['rg0_w1a']), padw(p['rg0_wp'])], axis=1)
    b1 = jnp.concatenate([p['rg0_ba'], jnp.zeros((256,), _f32)]).reshape(1, 384)
    a0, z0, sc0 = _t1(vf, w1, b1)

    seg = lambda z: _segsum(z, srcp, dstp, zeros128)
    cat2 = lambda wa, wb_: jnp.concatenate([wa, wb_], axis=1)
    bias2 = lambda bb: jnp.concatenate(
        [bb, jnp.zeros((128,), _f32)]).reshape(1, 256)

    P = seg(z0)
    a1, z1 = _tmid(a0, P[0], P[1], d0, d1, None,
                   cat2(p['rg0_w0b'], p['rg0_w1b']), bias2(p['rg0_bb']),
                   relu=True, emit_x=False)
    P = seg(z1)
    f0, a2, z2 = _tmid(a1, P[0], P[1], d0, d1, sc0,
                       cat2(p['rg1_w0a'], p['rg1_w1a']), bias2(p['rg1_ba']),
                       relu=False, emit_x=True)
    P = seg(z2)
    a3, z3 = _tmid(a2, P[0], P[1], d0, d1, None,
                   cat2(p['rg1_w0b'], p['rg1_w1b']), bias2(p['rg1_bb']),
                   relu=True, emit_x=False)
    P = seg(z3)
    f1, a4, z4 = _tmid(a3, P[0], P[1], d0, d1, f0,
                       cat2(p['rg2_w0a'], p['rg2_w1a']), bias2(p['rg2_ba']),
                       relu=False, emit_x=True)
    P = seg(z4)
    a5, z5 = _tmid(a4, P[0], P[1], d0, d1, None,
                   cat2(p['rg2_w0b'], p['rg2_w1b']), bias2(p['rg2_bb']),
                   relu=True, emit_x=False)
    P = seg(z5)
    wg = jnp.zeros((128, 256), _f32)
    wg = wg.at[:, 0:3].set(p['gcf_w0']).at[:, 128:131].set(p['gcf_w1'])
    bg = jnp.zeros((256,), _f32).at[0:3].set(p['gcf_b']).reshape(1, 256)
    f2, a6, z6 = _tmid(a5, P[0], P[1], d0, d1, f1, wg, bg,
                       relu=False, emit_x=True)
    P = seg(z6)
    pos_pad = jnp.pad(pos_p, ((0, 0), (0, 125)))
    pos_out = _tfinal(pos_pad, a6, P[0], P[1], d0, d1)

    return (pos_out[:N, 0:3], f2[:N])


# speedup vs baseline: 3.4043x; 3.4043x over previous
import functools

import jax
import jax.numpy as jnp
from jax import lax
from jax.experimental import pallas as pl
from jax.experimental.pallas import tpu as pltpu
from jax.experimental.pallas import tpu_sc as plsc

_f32 = jnp.float32
_i32 = jnp.int32

N = 10000
NP = 10240
E = 60000
EP = 61440
NW = 32
VPW = NP // NW
EPW = EP // NW
ECH = 128
CV = 32
ROWS_PER_TILE = NP // 16

LVL = [(56, 0), (28, 3136), (14, 3920), (7, 4120)]
T_ROWS = 4176

_MESH = plsc.VectorSubcoreMesh(core_axis_name="c", subcore_axis_name="s",
                               num_cores=2, num_subcores=16)


def _gbuild(f0t, f1t, f2t, f3t, w_lin, b_lin):
    def body(f0_ref, f1_ref, f2_ref, f3_ref, w_ref, b_ref, out_ref):
        qb = b_ref[...] * 0.25
        out_ref[0:3136, :] = jnp.dot(f0_ref[...], w_ref[0:256, :],
                                     preferred_element_type=_f32) + qb
        out_ref[3136:3920, :] = jnp.dot(f1_ref[...], w_ref[256:768, :],
                                        preferred_element_type=_f32) + qb
        out_ref[3920:4116, :] = jnp.dot(f2_ref[...], w_ref[768:1792, :],
                                        preferred_element_type=_f32) + qb
        out_ref[4120:4169, :] = jnp.dot(f3_ref[...], w_ref[1792:3840, :],
                                        preferred_element_type=_f32) + qb

    return pl.pallas_call(
        body,
        out_shape=jax.ShapeDtypeStruct((T_ROWS, 128), _f32),
    )(f0t, f1t, f2t, f3t, w_lin, b_lin)


def _align_body(g_hbm, px_hbm, py_hbm, pz_hbm, out_hbm,
                xv, yv, zv, idxb, wb, rows, outb, sem):
    c = lax.axis_index("c")
    s = lax.axis_index("s")
    wid = s * 2 + c
    vbase = wid * VPW
    pltpu.sync_copy(px_hbm.at[pl.ds(vbase, VPW)], xv)
    pltpu.sync_copy(py_hbm.at[pl.ds(vbase, VPW)], yv)
    pltpu.sync_copy(pz_hbm.at[pl.ds(vbase, VPW)], zv)

    def build(k, carry):
        sl = pl.ds(k * 16, 16)
        x = xv[sl]
        y = yv[sl]
        z = zv[sl]
        d = jnp.abs(z) + 2.0
        u = jnp.clip(x / d, -1.0, 1.0)
        v = jnp.clip(y / d, -1.0, 1.0)
        for l, (wl, off) in enumerate(LVL):
            fw = float(wl - 1)
            pxx = (u + 1.0) * 0.5 * fw
            pyy = (v + 1.0) * 0.5 * fw
            x0 = pxx.astype(_i32)
            y0 = pyy.astype(_i32)
            dx = jnp.minimum(x0 + 1, wl - 1) - x0
            y1 = jnp.minimum(y0 + 1, wl - 1)
            wx = pxx - x0.astype(_f32)
            wy = pyy - y0.astype(_f32)
            r0 = y0 * wl + (x0 + off)
            r1 = y1 * wl + (x0 + off)
            idxb[4 * l + 0, sl] = r0
            idxb[4 * l + 1, sl] = r0 + dx
            idxb[4 * l + 2, sl] = r1
            idxb[4 * l + 3, sl] = r1 + dx
            wb[pl.ds((4 * l + 0) * VPW + k * 16, 16)] = (1.0 - wx) * (1.0 - wy)
            wb[pl.ds((4 * l + 1) * VPW + k * 16, 16)] = wx * (1.0 - wy)
            wb[pl.ds((4 * l + 2) * VPW + k * 16, 16)] = (1.0 - wx) * wy
            wb[pl.ds((4 * l + 3) * VPW + k * 16, 16)] = wx * wy
        return carry

    lax.fori_loop(0, VPW // 16, build, 0)

    for ci in range(VPW // CV):
        cb = ci * CV
        handles = [
            pltpu.async_copy(g_hbm.at[idxb.at[t, pl.ds(cb, CV)]], rows.at[t], sem)
            for t in range(16)
        ]
        for h in handles:
            h.wait()

        def accum(i, carry):
            accs = [jnp.zeros((16,), _f32) for _ in range(8)]
            col = jnp.full((16,), cb + i, _i32)
            for t in range(16):
                wsp = plsc.load_gather(wb, [col + (t * VPW)])
                for b in range(8):
                    accs[b] = accs[b] + wsp * rows[t, i, pl.ds(b * 16, 16)]
            for b in range(8):
                outb[i, pl.ds(b * 16, 16)] = accs[b]
            return carry

        lax.fori_loop(0, CV, accum, 0)
        pltpu.sync_copy(outb, out_hbm.at[pl.ds(vbase + cb, CV)])


_align = functools.partial(
    pl.kernel,
    _align_body,
    out_type=jax.ShapeDtypeStruct((NP, 128), _f32),
    mesh=_MESH,
    compiler_params=pltpu.CompilerParams(needs_layout_passes=False),
    scratch_types=[
        pltpu.VMEM((VPW,), _f32),
        pltpu.VMEM((VPW,), _f32),
        pltpu.VMEM((VPW,), _f32),
        pltpu.VMEM((16, VPW), _i32),
        pltpu.VMEM((16 * VPW,), _f32),
        pltpu.VMEM((16, CV, 128), _f32),
        pltpu.VMEM((CV, 128), _f32),
        pltpu.SemaphoreType.DMA,
    ],
)()


def _segsum_body(z_hbm, srcp_hbm, dstp_hbm, zeros_hbm, out_hbm,
                 sidx, didx, rows, acc_sh, sem):
    c = lax.axis_index("c")
    s = lax.axis_index("s")
    wid = s * 2 + c
    pltpu.sync_copy(zeros_hbm,
                    acc_sh.at[pl.ds(s * ROWS_PER_TILE, ROWS_PER_TILE)])
    plsc.subcore_barrier()
    base = wid * EPW
    for k in range(EPW // ECH):
        pltpu.sync_copy(srcp_hbm.at[pl.ds(base + k * ECH, ECH)], sidx)
        pltpu.async_copy(z_hbm.at[sidx], rows, sem).wait()
        pltpu.sync_copy(dstp_hbm.at[pl.ds(base + k * ECH, ECH)], didx)
        pltpu.sync_copy(rows, acc_sh.at[didx], add=True)
    plsc.subcore_barrier()
    pltpu.sync_copy(acc_sh.at[pl.ds(s * ROWS_PER_TILE, ROWS_PER_TILE)],
                    out_hbm.at[c, pl.ds(s * ROWS_PER_TILE, ROWS_PER_TILE)])


_segsum = functools.partial(
    pl.kernel,
    _segsum_body,
    out_type=jax.ShapeDtypeStruct((2, NP, 128), _f32),
    mesh=_MESH,
    compiler_params=pltpu.CompilerParams(needs_layout_passes=False),
    scratch_types=[
        pltpu.VMEM((ECH,), _i32),
        pltpu.VMEM((ECH,), _i32),
        pltpu.VMEM((ECH, 128), _f32),
        pltpu.VMEM_SHARED((NP, 128), _f32),
        pltpu.SemaphoreType.DMA,
    ],
)()


_BM = 2048


def _t1(vf, w, b):
    def body(vf_ref, w_ref, b_ref, a_ref, z_ref, s_ref):
        az = jnp.dot(vf_ref[...], w_ref[...],
                     preferred_element_type=_f32) + b_ref[...]
        a_ref[...] = az[:, 0:128]
        z_ref[...] = az[:, 128:256]
        s_ref[...] = az[:, 256:384]

    return pl.pallas_call(
        body,
        grid=(NP // _BM,),
        in_specs=[
            pl.BlockSpec((_BM, 384), lambda i: (i, 0)),
            pl.BlockSpec((384, 384), lambda i: (0, 0)),
            pl.BlockSpec((1, 384), lambda i: (0, 0)),
        ],
        out_specs=[pl.BlockSpec((_BM, 128), lambda i: (i, 0))] * 3,
        out_shape=[jax.ShapeDtypeStruct((NP, 128), _f32)] * 3,
    )(vf, w, b)


def _nbr_of(p0, p1, d0, d1):
    deg = jnp.maximum(d0[:, 0:1] + d1[:, 0:1], 1.0)
    return (p0 + p1) / deg


def _tmid(aprev, p0, p1, d0, d1, resid, w, b, *, relu, emit_x):
    has_resid = resid is not None

    def body(*refs):
        if has_resid:
            (ap_ref, p0_ref, p1_ref, d0_ref, d1_ref, r_ref, w_ref, b_ref,
             *outs) = refs
        else:
            (ap_ref, p0_ref, p1_ref, d0_ref, d1_ref, w_ref, b_ref,
             *outs) = refs
        x = ap_ref[...] + _nbr_of(p0_ref[...], p1_ref[...],
                                  d0_ref[...], d1_ref[...])
        if relu:
            x = jnp.maximum(x, 0.0)
        if has_resid:
            x = x + r_ref[...]
        az = jnp.dot(x, w_ref[...], preferred_element_type=_f32) + b_ref[...]
        if emit_x:
            outs[0][...] = x
            outs = outs[1:]
        outs[0][...] = az[:, 0:128]
        outs[1][...] = az[:, 128:256]

    row = pl.BlockSpec((_BM, 128), lambda i: (i, 0))
    deg_spec = pl.BlockSpec((_BM, 16), lambda i: (i, 0))
    in_specs = [row, row, row, deg_spec, deg_spec]
    args = [aprev, p0, p1, d0, d1]
    if has_resid:
        in_specs.append(row)
        args.append(resid)
    in_specs += [pl.BlockSpec((128, 256), lambda i: (0, 0)),
                 pl.BlockSpec((1, 256), lambda i: (0, 0))]
    args += [w, b]
    n_out = 3 if emit_x else 2
    return pl.pallas_call(
        body,
        grid=(NP // _BM,),
        in_specs=in_specs,
        out_specs=[row] * n_out,
        out_shape=[jax.ShapeDtypeStruct((NP, 128), _f32)] * n_out,
    )(*args)


def _tfinal(pos_pad, a, p0, p1, d0, d1):
    def body(pos_ref, a_ref, p0_ref, p1_ref, d0_ref, d1_ref, out_ref):
        nbr = _nbr_of(p0_ref[...], p1_ref[...], d0_ref[...], d1_ref[...])
        out_ref[...] = pos_ref[...] + jnp.tanh(a_ref[...] + nbr)

    row = pl.BlockSpec((_BM, 128), lambda i: (i, 0))
    deg_spec = pl.BlockSpec((_BM, 16), lambda i: (i, 0))
    return pl.pallas_call(
        body,
        grid=(NP // _BM,),
        in_specs=[row, row, row, row, deg_spec, deg_spec],
        out_specs=row,
        out_shape=jax.ShapeDtypeStruct((NP, 128), _f32),
    )(pos_pad, a, p0, p1, d0, d1)


def kernel(conv2_3, conv3_4, conv4_6, conv5_3, vertex_positions,
           vertex_features, edge_index, params):
    p = params

    f0t = conv2_3.reshape(256, 3136).T
    f1t = conv3_4.reshape(512, 784).T
    f2t = conv4_6.reshape(1024, 196).T
    f3t = conv5_3.reshape(2048, 49).T
    g = _gbuild(f0t, f1t, f2t, f3t, p['W_lin'], p['b_lin'].reshape(1, 128))

    pos_p = jnp.pad(vertex_positions, ((0, NP - N), (0, 0)))
    proj = _align(g, pos_p[:, 0], pos_p[:, 1], pos_p[:, 2])

    srcp = jnp.concatenate([edge_index[0], jnp.zeros((EP - E,), _i32)])
    dstp = jnp.concatenate([edge_index[1], jnp.full((EP - E,), NP - 1, _i32)])
    zeros128 = jnp.zeros((ROWS_PER_TILE, 128), _f32)

    degp = _segsum(jnp.ones((NP, 128), _f32), srcp, dstp, zeros128)
    d0, d1 = degp[0][:, 0:16], degp[1][:, 0:16]

    vf_p = jnp.pad(vertex_features, ((0, NP - N), (0, 0)))
    vf = jnp.concatenate(
        [vf_p, pos_p, proj, jnp.zeros((NP, 125), _f32)], axis=1)

    padw = lambda w: jnp.pad(w, ((0, 384 - w.shape[0]), (0, 0)))
    w1 = jnp.concatenate(
        [padw(p['rg0_w0a']), padw(p['rg0_w1a']), padw(p['rg0_wp'])], axis=1)
    b1 = jnp.concatenate([p['rg0_ba'], jnp.zeros((256,), _f32)]).reshape(1, 384)
    a0, z0, sc0 = _t1(vf, w1, b1)

    seg = lambda z: _segsum(z, srcp, dstp, zeros128)
    cat2 = lambda wa, wb_: jnp.concatenate([wa, wb_], axis=1)
    bias2 = lambda bb: jnp.concatenate(
        [bb, jnp.zeros((128,), _f32)]).reshape(1, 256)

    P = seg(z0)
    a1, z1 = _tmid(a0, P[0], P[1], d0, d1, None,
                   cat2(p['rg0_w0b'], p['rg0_w1b']), bias2(p['rg0_bb']),
                   relu=True, emit_x=False)
    P = seg(z1)
    f0, a2, z2 = _tmid(a1, P[0], P[1], d0, d1, sc0,
                       cat2(p['rg1_w0a'], p['rg1_w1a']), bias2(p['rg1_ba']),
                       relu=False, emit_x=True)
    P = seg(z2)
    a3, z3 = _tmid(a2, P[0], P[1], d0, d1, None,
                   cat2(p['rg1_w0b'], p['rg1_w1b']), bias2(p['rg1_bb']),
                   relu=True, emit_x=False)
    P = seg(z3)
    f1, a4, z4 = _tmid(a3, P[0], P[1], d0, d1, f0,
                       cat2(p['rg2_w0a'], p['rg2_w1a']), bias2(p['rg2_ba']),
                       relu=False, emit_x=True)
    P = seg(z4)
    a5, z5 = _tmid(a4, P[0], P[1], d0, d1, None,
                   cat2(p['rg2_w0b'], p['rg2_w1b']), bias2(p['rg2_bb']),
                   relu=True, emit_x=False)
    P = seg(z5)
    wg = jnp.zeros((128, 256), _f32)
    wg = wg.at[:, 0:3].set(p['gcf_w0']).at[:, 128:131].set(p['gcf_w1'])
    bg = jnp.zeros((256,), _f32).at[0:3].set(p['gcf_b']).reshape(1, 256)
    f2, a6, z6 = _tmid(a5, P[0], P[1], d0, d1, f1, wg, bg,
                       relu=False, emit_x=True)
    P = seg(z6)
    pos_pad = jnp.pad(pos_p, ((0, 0), (0, 125)))
    pos_out = _tfinal(pos_pad, a6, P[0], P[1], d0, d1)

    return (pos_out[:N, 0:3], f2[:N])


# speedup vs baseline: 3.8474x; 1.1302x over previous
import functools

import jax
import jax.numpy as jnp
from jax import lax
from jax.experimental import pallas as pl
from jax.experimental.pallas import tpu as pltpu
from jax.experimental.pallas import tpu_sc as plsc

_f32 = jnp.float32
_i32 = jnp.int32

N = 10000
NP = 10240
E = 60000
EP = 61440
NW = 32
VPW = NP // NW
EPW = EP // NW
ECH = 64
CV = 32
ROWS_PER_TILE = NP // 16

LVL = [(56, 0), (28, 3136), (14, 3920), (7, 4120)]
T_ROWS = 4176

_MESH = plsc.VectorSubcoreMesh(core_axis_name="c", subcore_axis_name="s",
                               num_cores=2, num_subcores=16)


def _gbuild(f0t, f1t, f2t, f3t, w_lin, b_lin):
    def body(f0_ref, f1_ref, f2_ref, f3_ref, w_ref, b_ref, out_ref):
        qb = b_ref[...] * 0.25
        out_ref[0:3136, :] = jnp.dot(f0_ref[...], w_ref[0:256, :],
                                     preferred_element_type=_f32) + qb
        out_ref[3136:3920, :] = jnp.dot(f1_ref[...], w_ref[256:768, :],
                                        preferred_element_type=_f32) + qb
        out_ref[3920:4116, :] = jnp.dot(f2_ref[...], w_ref[768:1792, :],
                                        preferred_element_type=_f32) + qb
        out_ref[4120:4169, :] = jnp.dot(f3_ref[...], w_ref[1792:3840, :],
                                        preferred_element_type=_f32) + qb

    return pl.pallas_call(
        body,
        out_shape=jax.ShapeDtypeStruct((T_ROWS, 128), _f32),
    )(f0t, f1t, f2t, f3t, w_lin, b_lin)


def _align_body(g_hbm, px_hbm, py_hbm, pz_hbm, out_hbm,
                xv, yv, zv, idxb, wb, rows, outb, sem):
    c = lax.axis_index("c")
    s = lax.axis_index("s")
    wid = s * 2 + c
    vbase = wid * VPW
    pltpu.sync_copy(px_hbm.at[pl.ds(vbase, VPW)], xv)
    pltpu.sync_copy(py_hbm.at[pl.ds(vbase, VPW)], yv)
    pltpu.sync_copy(pz_hbm.at[pl.ds(vbase, VPW)], zv)

    def build(k, carry):
        sl = pl.ds(k * 16, 16)
        x = xv[sl]
        y = yv[sl]
        z = zv[sl]
        d = jnp.abs(z) + 2.0
        u = jnp.clip(x / d, -1.0, 1.0)
        v = jnp.clip(y / d, -1.0, 1.0)
        for l, (wl, off) in enumerate(LVL):
            fw = float(wl - 1)
            pxx = (u + 1.0) * 0.5 * fw
            pyy = (v + 1.0) * 0.5 * fw
            x0 = pxx.astype(_i32)
            y0 = pyy.astype(_i32)
            dx = jnp.minimum(x0 + 1, wl - 1) - x0
            y1 = jnp.minimum(y0 + 1, wl - 1)
            wx = pxx - x0.astype(_f32)
            wy = pyy - y0.astype(_f32)
            r0 = y0 * wl + (x0 + off)
            r1 = y1 * wl + (x0 + off)
            idxb[4 * l + 0, sl] = r0
            idxb[4 * l + 1, sl] = r0 + dx
            idxb[4 * l + 2, sl] = r1
            idxb[4 * l + 3, sl] = r1 + dx
            wb[pl.ds((4 * l + 0) * VPW + k * 16, 16)] = (1.0 - wx) * (1.0 - wy)
            wb[pl.ds((4 * l + 1) * VPW + k * 16, 16)] = wx * (1.0 - wy)
            wb[pl.ds((4 * l + 2) * VPW + k * 16, 16)] = (1.0 - wx) * wy
            wb[pl.ds((4 * l + 3) * VPW + k * 16, 16)] = wx * wy
        return carry

    lax.fori_loop(0, VPW // 16, build, 0)

    for ci in range(VPW // CV):
        cb = ci * CV
        handles = [
            pltpu.async_copy(g_hbm.at[idxb.at[t, pl.ds(cb, CV)]], rows.at[t], sem)
            for t in range(16)
        ]
        for h in handles:
            h.wait()

        def accum(i, carry):
            accs = [jnp.zeros((16,), _f32) for _ in range(8)]
            col = jnp.full((16,), cb + i, _i32)
            for t in range(16):
                wsp = plsc.load_gather(wb, [col + (t * VPW)])
                for b in range(8):
                    accs[b] = accs[b] + wsp * rows[t, i, pl.ds(b * 16, 16)]
            for b in range(8):
                outb[i, pl.ds(b * 16, 16)] = accs[b]
            return carry

        lax.fori_loop(0, CV, accum, 0)
        pltpu.sync_copy(outb, out_hbm.at[pl.ds(vbase + cb, CV)])


_align = functools.partial(
    pl.kernel,
    _align_body,
    out_type=jax.ShapeDtypeStruct((NP, 128), _f32),
    mesh=_MESH,
    compiler_params=pltpu.CompilerParams(needs_layout_passes=False),
    scratch_types=[
        pltpu.VMEM((VPW,), _f32),
        pltpu.VMEM((VPW,), _f32),
        pltpu.VMEM((VPW,), _f32),
        pltpu.VMEM((16, VPW), _i32),
        pltpu.VMEM((16 * VPW,), _f32),
        pltpu.VMEM((16, CV, 128), _f32),
        pltpu.VMEM((CV, 128), _f32),
        pltpu.SemaphoreType.DMA,
    ],
)()


KCH = EPW // ECH
RING = 5


def _segsum_body(z_hbm, src2_hbm, dst2_hbm, zeros_hbm, out_hbm,
                 sidx, didx, rows, acc_sh, sem_g, sem_s):
    c = lax.axis_index("c")
    s = lax.axis_index("s")
    wid = s * 2 + c
    pltpu.sync_copy(src2_hbm.at[wid], sidx)
    pltpu.sync_copy(dst2_hbm.at[wid], didx)
    gh = [None] * KCH
    sh = [None] * KCH
    for k in range(RING - 1):
        gh[k] = pltpu.async_copy(z_hbm.at[sidx.at[k]], rows.at[k % RING], sem_g)
    pltpu.sync_copy(zeros_hbm,
                    acc_sh.at[pl.ds(s * ROWS_PER_TILE, ROWS_PER_TILE)])
    plsc.subcore_barrier()
    for k in range(KCH):
        gh[k].wait()
        sh[k] = pltpu.async_copy(rows.at[k % RING], acc_sh.at[didx.at[k]],
                                 sem_s, add=True)
        nk = k + RING - 1
        if nk < KCH:
            if k > 0:
                sh[k - 1].wait()
            gh[nk] = pltpu.async_copy(z_hbm.at[sidx.at[nk]],
                                      rows.at[nk % RING], sem_g)
    for k in range(max(KCH - RING, 0), KCH):
        sh[k].wait()
    plsc.subcore_barrier()
    pltpu.sync_copy(acc_sh.at[pl.ds(s * ROWS_PER_TILE, ROWS_PER_TILE)],
                    out_hbm.at[c, pl.ds(s * ROWS_PER_TILE, ROWS_PER_TILE)])


_segsum = functools.partial(
    pl.kernel,
    _segsum_body,
    out_type=jax.ShapeDtypeStruct((2, NP, 128), _f32),
    mesh=_MESH,
    compiler_params=pltpu.CompilerParams(needs_layout_passes=False),
    scratch_types=[
        pltpu.VMEM((KCH, ECH), _i32),
        pltpu.VMEM((KCH, ECH), _i32),
        pltpu.VMEM((RING, ECH, 128), _f32),
        pltpu.VMEM_SHARED((NP, 128), _f32),
        pltpu.SemaphoreType.DMA,
        pltpu.SemaphoreType.DMA,
    ],
)()


_BM = 2048


def _t1(vf, w, b):
    def body(vf_ref, w_ref, b_ref, a_ref, z_ref, s_ref):
        az = jnp.dot(vf_ref[...], w_ref[...],
                     preferred_element_type=_f32) + b_ref[...]
        a_ref[...] = az[:, 0:128]
        z_ref[...] = az[:, 128:256]
        s_ref[...] = az[:, 256:384]

    return pl.pallas_call(
        body,
        grid=(NP // _BM,),
        in_specs=[
            pl.BlockSpec((_BM, 384), lambda i: (i, 0)),
            pl.BlockSpec((384, 384), lambda i: (0, 0)),
            pl.BlockSpec((1, 384), lambda i: (0, 0)),
        ],
        out_specs=[pl.BlockSpec((_BM, 128), lambda i: (i, 0))] * 3,
        out_shape=[jax.ShapeDtypeStruct((NP, 128), _f32)] * 3,
    )(vf, w, b)


def _nbr_of(p0, p1, d0, d1):
    deg = jnp.maximum(d0[:, 0:1] + d1[:, 0:1], 1.0)
    return (p0 + p1) / deg


def _tmid(aprev, p0, p1, d0, d1, resid, w, b, *, relu, emit_x):
    has_resid = resid is not None

    def body(*refs):
        if has_resid:
            (ap_ref, p0_ref, p1_ref, d0_ref, d1_ref, r_ref, w_ref, b_ref,
             *outs) = refs
        else:
            (ap_ref, p0_ref, p1_ref, d0_ref, d1_ref, w_ref, b_ref,
             *outs) = refs
        x = ap_ref[...] + _nbr_of(p0_ref[...], p1_ref[...],
                                  d0_ref[...], d1_ref[...])
        if relu:
            x = jnp.maximum(x, 0.0)
        if has_resid:
            x = x + r_ref[...]
        az = jnp.dot(x, w_ref[...], preferred_element_type=_f32) + b_ref[...]
        if emit_x:
            outs[0][...] = x
            outs = outs[1:]
        outs[0][...] = az[:, 0:128]
        outs[1][...] = az[:, 128:256]

    row = pl.BlockSpec((_BM, 128), lambda i: (i, 0))
    deg_spec = pl.BlockSpec((_BM, 16), lambda i: (i, 0))
    in_specs = [row, row, row, deg_spec, deg_spec]
    args = [aprev, p0, p1, d0, d1]
    if has_resid:
        in_specs.append(row)
        args.append(resid)
    in_specs += [pl.BlockSpec((128, 256), lambda i: (0, 0)),
                 pl.BlockSpec((1, 256), lambda i: (0, 0))]
    args += [w, b]
    n_out = 3 if emit_x else 2
    return pl.pallas_call(
        body,
        grid=(NP // _BM,),
        in_specs=in_specs,
        out_specs=[row] * n_out,
        out_shape=[jax.ShapeDtypeStruct((NP, 128), _f32)] * n_out,
    )(*args)


def _tfinal(pos_pad, a, p0, p1, d0, d1):
    def body(pos_ref, a_ref, p0_ref, p1_ref, d0_ref, d1_ref, out_ref):
        nbr = _nbr_of(p0_ref[...], p1_ref[...], d0_ref[...], d1_ref[...])
        out_ref[...] = pos_ref[...] + jnp.tanh(a_ref[...] + nbr)

    row = pl.BlockSpec((_BM, 128), lambda i: (i, 0))
    deg_spec = pl.BlockSpec((_BM, 16), lambda i: (i, 0))
    return pl.pallas_call(
        body,
        grid=(NP // _BM,),
        in_specs=[row, row, row, row, deg_spec, deg_spec],
        out_specs=row,
        out_shape=jax.ShapeDtypeStruct((NP, 128), _f32),
    )(pos_pad, a, p0, p1, d0, d1)


def kernel(conv2_3, conv3_4, conv4_6, conv5_3, vertex_positions,
           vertex_features, edge_index, params):
    p = params

    f0t = conv2_3.reshape(256, 3136).T
    f1t = conv3_4.reshape(512, 784).T
    f2t = conv4_6.reshape(1024, 196).T
    f3t = conv5_3.reshape(2048, 49).T
    g = _gbuild(f0t, f1t, f2t, f3t, p['W_lin'], p['b_lin'].reshape(1, 128))

    pos_p = jnp.pad(vertex_positions, ((0, NP - N), (0, 0)))
    proj = _align(g, pos_p[:, 0], pos_p[:, 1], pos_p[:, 2])

    srcp = jnp.concatenate(
        [edge_index[0], jnp.zeros((EP - E,), _i32)]).reshape(NW, KCH, ECH)
    dstp = jnp.concatenate(
        [edge_index[1], jnp.full((EP - E,), NP - 1, _i32)]).reshape(NW, KCH, ECH)
    zeros128 = jnp.zeros((ROWS_PER_TILE, 128), _f32)

    degp = _segsum(jnp.ones((NP, 128), _f32), srcp, dstp, zeros128)
    d0, d1 = degp[0][:, 0:16], degp[1][:, 0:16]

    vf_p = jnp.pad(vertex_features, ((0, NP - N), (0, 0)))
    vf = jnp.concatenate(
        [vf_p, pos_p, proj, jnp.zeros((NP, 125), _f32)], axis=1)

    padw = lambda w: jnp.pad(w, ((0, 384 - w.shape[0]), (0, 0)))
    w1 = jnp.concatenate(
        [padw(p['rg0_w0a']), padw(p['rg0_w1a']), padw(p['rg0_wp'])], axis=1)
    b1 = jnp.concatenate([p['rg0_ba'], jnp.zeros((256,), _f32)]).reshape(1, 384)
    a0, z0, sc0 = _t1(vf, w1, b1)

    seg = lambda z: _segsum(z, srcp, dstp, zeros128)
    cat2 = lambda wa, wb_: jnp.concatenate([wa, wb_], axis=1)
    bias2 = lambda bb: jnp.concatenate(
        [bb, jnp.zeros((128,), _f32)]).reshape(1, 256)

    P = seg(z0)
    a1, z1 = _tmid(a0, P[0], P[1], d0, d1, None,
                   cat2(p['rg0_w0b'], p['rg0_w1b']), bias2(p['rg0_bb']),
                   relu=True, emit_x=False)
    P = seg(z1)
    f0, a2, z2 = _tmid(a1, P[0], P[1], d0, d1, sc0,
                       cat2(p['rg1_w0a'], p['rg1_w1a']), bias2(p['rg1_ba']),
                       relu=False, emit_x=True)
    P = seg(z2)
    a3, z3 = _tmid(a2, P[0], P[1], d0, d1, None,
                   cat2(p['rg1_w0b'], p['rg1_w1b']), bias2(p['rg1_bb']),
                   relu=True, emit_x=False)
    P = seg(z3)
    f1, a4, z4 = _tmid(a3, P[0], P[1], d0, d1, f0,
                       cat2(p['rg2_w0a'], p['rg2_w1a']), bias2(p['rg2_ba']),
                       relu=False, emit_x=True)
    P = seg(z4)
    a5, z5 = _tmid(a4, P[0], P[1], d0, d1, None,
                   cat2(p['rg2_w0b'], p['rg2_w1b']), bias2(p['rg2_bb']),
                   relu=True, emit_x=False)
    P = seg(z5)
    wg = jnp.zeros((128, 256), _f32)
    wg = wg.at[:, 0:3].set(p['gcf_w0']).at[:, 128:131].set(p['gcf_w1'])
    bg = jnp.zeros((256,), _f32).at[0:3].set(p['gcf_b']).reshape(1, 256)
    f2, a6, z6 = _tmid(a5, P[0], P[1], d0, d1, f1, wg, bg,
                       relu=False, emit_x=True)
    P = seg(z6)
    pos_pad = jnp.pad(pos_p, ((0, 0), (0, 125)))
    pos_out = _tfinal(pos_pad, a6, P[0], P[1], d0, d1)

    return (pos_out[:N, 0:3], f2[:N])
